# baseline (speedup 1.0000x reference)
import functools

import jax
import jax.numpy as jnp
from jax import lax
from jax.experimental import pallas as pl
from jax.experimental.pallas import tpu as pltpu
from jax.experimental.pallas import tpu_sc as plsc

N = 10000
E = 320000
D_IN = 128
D_HID = 256
D_OUT = 128

NC = 2
NS = 16

EP = 327680
ER = EP // 128
PAD = EP - E
NP = N + 16
RPT = ER // NS
RPW = ER // (NC * NS)

_mesh = plsc.VectorSubcoreMesh(core_axis_name="c", subcore_axis_name="s")

@functools.partial(
    pl.kernel,
    out_type=jax.ShapeDtypeStruct((NC, NP), jnp.float32),
    mesh=_mesh,
    scratch_types=[
        pltpu.VMEM((8, 128), jnp.int32),
        pltpu.VMEM((128,), jnp.float32),
        pltpu.VMEM_SHARED((NP,), jnp.float32),
    ],
)
def _deg_kernel(ones_hbm, col2_hbm, deg_hbm, cidx, ones_v, dacc):
    c = lax.axis_index("c")
    s = lax.axis_index("s")
    w = s * NC + c

    @pl.when(s == 0)
    def _():
        pltpu.sync_copy(ones_hbm, dacc)

    pltpu.sync_copy(ones_hbm.at[pl.ds(0, 128)], ones_v)
    plsc.subcore_barrier()

    def body(o, carry):
        rb = w * RPW + o * 8
        pltpu.sync_copy(col2_hbm.at[pl.ds(rb, 8)], cidx)
        for j in range(8):
            pltpu.sync_copy(ones_v, dacc.at[cidx.at[j]], add=True)
        return carry

    lax.fori_loop(0, RPW // 8, body, 0)
    plsc.subcore_barrier()

    @pl.when(s == 0)
    def _():
        pltpu.sync_copy(dacc, deg_hbm.at[c])


def _edge_pipeline(g_hbm, acc, ridx_src, cidx_src, base, ibufs, bufs,
                   gsems, ssems, irsems, icsems, nb):
    rbuf, cbuf = ibufs

    def iload(b, par, sync):
        rsrc = ridx_src.at[pl.ds(base + b * 8, 8)]
        csrc = cidx_src.at[pl.ds(base + b * 8, 8)]
        if sync:
            pltpu.sync_copy(rsrc, rbuf[par])
            pltpu.sync_copy(csrc, cbuf[par])
        else:
            pltpu.async_copy(rsrc, rbuf[par], irsems[par])
            pltpu.async_copy(csrc, cbuf[par], icsems[par])

    def iwait(par):
        pltpu.make_async_copy(ridx_src.at[pl.ds(0, 8)], rbuf[par], irsems[par]).wait()
        pltpu.make_async_copy(cidx_src.at[pl.ds(0, 8)], cbuf[par], icsems[par]).wait()

    def gwait(p):
        pltpu.make_async_copy(g_hbm.at[rbuf[0].at[0]], bufs[p], gsems[p]).wait()

    def swait(p):
        pltpu.make_async_copy(bufs[p], acc.at[cbuf[0].at[0]], ssems[p]).wait()

    def gissue(ib, j, p):
        pltpu.async_copy(g_hbm.at[rbuf[ib].at[j]], bufs[p], gsems[p])

    def sissue(ib, j, p):
        pltpu.async_copy(bufs[p], acc.at[cbuf[ib].at[j]], ssems[p], add=True)

    iload(0, 0, sync=True)
    gissue(0, 0, 0)

    def body(m, carry):
        for pb in range(2):
            b = 2 * m + pb
            for j in range(8):
                p = j % 2
                gwait(p)
                if j == 0:
                    @pl.when(b > 0)
                    def _():
                        swait(1)
                else:
                    swait(1 - p)
                if j < 7:
                    gissue(pb, j + 1, 1 - p)
                else:
                    @pl.when(b + 1 < nb)
                    def _():
                        iwait(1 - pb)
                        gissue(1 - pb, 0, 0)
                if j == 2:
                    @pl.when(b + 1 < nb)
                    def _():
                        iload(b + 1, 1 - pb, sync=False)
                sissue(pb, j, p)
        return carry

    lax.fori_loop(0, nb // 2, body, 0)
    swait(1)


def _make_agg_kernel(dh):
    @functools.partial(
        pl.kernel,
        out_type=jax.ShapeDtypeStruct((2 * N, dh), jnp.float32),
        mesh=_mesh,
        scratch_types=[
            pltpu.VMEM((8, 128), jnp.int32),
            pltpu.VMEM((8, 128), jnp.int32),
            pltpu.VMEM((8, 128), jnp.int32),
            pltpu.VMEM((8, 128), jnp.int32),
            pltpu.VMEM((128, dh), jnp.float32),
            pltpu.VMEM((128, dh), jnp.float32),
            pltpu.VMEM_SHARED((NP, dh), jnp.float32),
            pltpu.SemaphoreType.DMA,
            pltpu.SemaphoreType.DMA,
            pltpu.SemaphoreType.DMA,
            pltpu.SemaphoreType.DMA,
            pltpu.SemaphoreType.DMA,
            pltpu.SemaphoreType.DMA,
            pltpu.SemaphoreType.DMA,
            pltpu.SemaphoreType.DMA,
        ],
    )
    def agg(g_hbm, rowpair_hbm, col2_hbm, out_hbm, ridx0, ridx1, cidx0, cidx1,
            rows0, rows1, acc, gsem0, gsem1, ssem0, ssem1,
            irsem0, irsem1, icsem0, icsem1):
        c = lax.axis_index("c")
        s = lax.axis_index("s")

        @pl.when(s < 15)
        def _():
            pltpu.sync_copy(
                g_hbm.at[pl.ds(c * N + s * 640, 640)], acc.at[pl.ds(s * 640, 640)]
            )

        @pl.when(s == 15)
        def _():
            pltpu.sync_copy(
                g_hbm.at[pl.ds(c * N + 9600, 400)], acc.at[pl.ds(9600, 400)]
            )

        plsc.subcore_barrier()

        _edge_pipeline(
            g_hbm, acc, rowpair_hbm.at[c], col2_hbm, s * RPT,
            ((ridx0, ridx1), (cidx0, cidx1)), (rows0, rows1),
            (gsem0, gsem1), (ssem0, ssem1), (irsem0, irsem1), (icsem0, icsem1),
            RPT // 8,
        )
        plsc.subcore_barrier()

        @pl.when(s < 15)
        def _():
            pltpu.sync_copy(
                acc.at[pl.ds(s * 640, 640)], out_hbm.at[pl.ds(c * N + s * 640, 640)]
            )

        @pl.when(s == 15)
        def _():
            pltpu.sync_copy(
                acc.at[pl.ds(9600, 400)], out_hbm.at[pl.ds(c * N + 9600, 400)]
            )

    return agg


_agg1 = _make_agg_kernel(D_HID // 2)


@functools.partial(
    pl.kernel,
    out_type=jax.ShapeDtypeStruct((NC, N, D_OUT), jnp.float32),
    mesh=_mesh,
    scratch_types=[
        pltpu.VMEM((8, 128), jnp.int32),
        pltpu.VMEM((8, 128), jnp.int32),
        pltpu.VMEM((8, 128), jnp.int32),
        pltpu.VMEM((8, 128), jnp.int32),
        pltpu.VMEM((128, D_OUT), jnp.float32),
        pltpu.VMEM((128, D_OUT), jnp.float32),
        pltpu.VMEM_SHARED((NP, D_OUT), jnp.float32),
        pltpu.SemaphoreType.DMA,
        pltpu.SemaphoreType.DMA,
        pltpu.SemaphoreType.DMA,
        pltpu.SemaphoreType.DMA,
        pltpu.SemaphoreType.DMA,
        pltpu.SemaphoreType.DMA,
        pltpu.SemaphoreType.DMA,
        pltpu.SemaphoreType.DMA,
    ],
)
def _agg2(g_hbm, row2_hbm, col2_hbm, out_hbm, ridx0, ridx1, cidx0, cidx1,
          rows0, rows1, acc, gsem0, gsem1, ssem0, ssem1,
          irsem0, irsem1, icsem0, icsem1):
    c = lax.axis_index("c")
    s = lax.axis_index("s")

    @pl.when(s < 15)
    def _():
        pltpu.sync_copy(g_hbm.at[pl.ds(s * 640, 640)], acc.at[pl.ds(s * 640, 640)])

    @pl.when(s == 15)
    def _():
        pltpu.sync_copy(g_hbm.at[pl.ds(9600, 400)], acc.at[pl.ds(9600, 400)])

    rpt = ER // (NC * NS)
    plsc.subcore_barrier()

    _edge_pipeline(
        g_hbm, acc, row2_hbm, col2_hbm, (s * NC + c) * rpt,
        ((ridx0, ridx1), (cidx0, cidx1)), (rows0, rows1),
        (gsem0, gsem1), (ssem0, ssem1), (irsem0, irsem1), (icsem0, icsem1),
        rpt // 8,
    )
    plsc.subcore_barrier()

    @pl.when(s < 15)
    def _():
        pltpu.sync_copy(acc.at[pl.ds(s * 640, 640)], out_hbm.at[c, pl.ds(s * 640, 640)])

    @pl.when(s == 15)
    def _():
        pltpu.sync_copy(acc.at[pl.ds(9600, 400)], out_hbm.at[c, pl.ds(9600, 400)])

_B = 1000
_NB = N // _B


def _tc0_body(x_ref, w1_ref, h1_ref):
    h1_ref[...] = jnp.dot(
        x_ref[...], w1_ref[...], preferred_element_type=jnp.float32
    )


def _tc1_body(pdeg_ref, h1_ref, dis_ref, g1_ref):
    deg = pdeg_ref[0] + pdeg_ref[1] - 1.0
    dis = lax.rsqrt(deg)
    dis_ref[...] = dis
    g = h1_ref[...] * dis
    g1_ref[0] = g[:, : D_HID // 2]
    g1_ref[1] = g[:, D_HID // 2 :]


def _tc2_body(agg1_ref, dis_ref, b1_ref, w2_ref, g2_ref):
    dis = dis_ref[...]
    o_l = jnp.maximum(agg1_ref[0] * dis + b1_ref[0, : D_HID // 2], 0.0)
    o_r = jnp.maximum(agg1_ref[1] * dis + b1_ref[0, D_HID // 2 :], 0.0)
    h2 = jnp.dot(o_l, w2_ref[: D_HID // 2], preferred_element_type=jnp.float32)
    h2 = h2 + jnp.dot(o_r, w2_ref[D_HID // 2 :], preferred_element_type=jnp.float32)
    g2_ref[...] = h2 * dis


def _tc3_body(agg2_ref, g2_ref, dis_ref, b2_ref, out_ref):
    full = agg2_ref[0] + agg2_ref[1] - g2_ref[...]
    out_ref[...] = full * dis_ref[...] + b2_ref[...]


_tc0 = pl.pallas_call(
    _tc0_body,
    grid=(_NB,),
    in_specs=[
        pl.BlockSpec((_B, D_IN), lambda i: (i, 0)),
        pl.BlockSpec((D_IN, D_HID), lambda i: (0, 0)),
    ],
    out_specs=pl.BlockSpec((_B, D_HID), lambda i: (i, 0)),
    out_shape=jax.ShapeDtypeStruct((N, D_HID), jnp.float32),
)

_tc1 = pl.pallas_call(
    _tc1_body,
    grid=(_NB,),
    in_specs=[
        pl.BlockSpec((2, _B, 1), lambda i: (0, i, 0)),
        pl.BlockSpec((_B, D_HID), lambda i: (i, 0)),
    ],
    out_specs=[
        pl.BlockSpec((_B, 1), lambda i: (i, 0)),
        pl.BlockSpec((2, _B, D_HID // 2), lambda i: (0, i, 0)),
    ],
    out_shape=[
        jax.ShapeDtypeStruct((N, 1), jnp.float32),
        jax.ShapeDtypeStruct((2, N, D_HID // 2), jnp.float32),
    ],
)

_tc2 = pl.pallas_call(
    _tc2_body,
    grid=(_NB,),
    in_specs=[
        pl.BlockSpec((2, _B, D_HID // 2), lambda i: (0, i, 0)),
        pl.BlockSpec((_B, 1), lambda i: (i, 0)),
        pl.BlockSpec((1, D_HID), lambda i: (0, 0)),
        pl.BlockSpec((D_HID, D_OUT), lambda i: (0, 0)),
    ],
    out_specs=pl.BlockSpec((_B, D_OUT), lambda i: (i, 0)),
    out_shape=jax.ShapeDtypeStruct((N, D_OUT), jnp.float32),
)

_tc3 = pl.pallas_call(
    _tc3_body,
    grid=(_NB,),
    in_specs=[
        pl.BlockSpec((2, _B, D_OUT), lambda i: (0, i, 0)),
        pl.BlockSpec((_B, D_OUT), lambda i: (i, 0)),
        pl.BlockSpec((_B, 1), lambda i: (i, 0)),
        pl.BlockSpec((1, D_OUT), lambda i: (0, 0)),
    ],
    out_specs=pl.BlockSpec((_B, D_OUT), lambda i: (i, 0)),
    out_shape=jax.ShapeDtypeStruct((N, D_OUT), jnp.float32),
)


def kernel(x, edge_index, W1, b1, W2, b2):
    row = edge_index[0]
    col = edge_index[1]
    pad_i = jnp.arange(PAD, dtype=jnp.int32)
    rowp = jnp.concatenate([row, pad_i % 64])
    colp = jnp.concatenate([col, N + (pad_i % 16)])
    col2 = colp.reshape(ER, 128)
    row2 = rowp.reshape(ER, 128)
    rowpair = jnp.stack([rowp, rowp + N]).reshape(2, ER, 128)
    ones = jnp.ones((NP,), jnp.float32)

    h1 = _tc0(x, W1)
    deg2 = _deg_kernel(ones, col2)
    pdeg = deg2[:, :N].reshape(2, N, 1)
    dis, g1 = _tc1(pdeg, h1)
    agg1 = _agg1(g1.reshape(2 * N, D_HID // 2), rowpair, col2)
    g2 = _tc2(agg1.reshape(2, N, D_HID // 2), dis, b1.reshape(1, D_HID), W2)
    agg2 = _agg2(g2, row2, col2)
    out = _tc3(agg2, g2, dis, b2.reshape(1, D_OUT))
    return out


# speedup vs baseline: 1.0246x; 1.0246x over previous
import functools

import jax
import jax.numpy as jnp
from jax import lax
from jax.experimental import pallas as pl
from jax.experimental.pallas import tpu as pltpu
from jax.experimental.pallas import tpu_sc as plsc

N = 10000
E = 320000
D_IN = 128
D_HID = 256
D_OUT = 128

NC = 2
NS = 16

EP = 327680
ER = EP // 128
PAD = EP - E
NP = N + 16
RPT = ER // NS
RPW = ER // (NC * NS)

_mesh = plsc.VectorSubcoreMesh(core_axis_name="c", subcore_axis_name="s")

@functools.partial(
    pl.kernel,
    out_type=jax.ShapeDtypeStruct((NC, NP), jnp.float32),
    mesh=_mesh,
    scratch_types=[
        pltpu.VMEM((8, 128), jnp.int32),
        pltpu.VMEM((128,), jnp.float32),
        pltpu.VMEM_SHARED((NP,), jnp.float32),
    ],
)
def _deg_kernel(ones_hbm, col2_hbm, deg_hbm, cidx, ones_v, dacc):
    c = lax.axis_index("c")
    s = lax.axis_index("s")
    w = s * NC + c

    @pl.when(s == 0)
    def _():
        pltpu.sync_copy(ones_hbm, dacc)

    pltpu.sync_copy(ones_hbm.at[pl.ds(0, 128)], ones_v)
    plsc.subcore_barrier()

    def body(o, carry):
        rb = w * RPW + o * 8
        pltpu.sync_copy(col2_hbm.at[pl.ds(rb, 8)], cidx)
        for j in range(8):
            pltpu.sync_copy(ones_v, dacc.at[cidx.at[j]], add=True)
        return carry

    lax.fori_loop(0, RPW // 8, body, 0)
    plsc.subcore_barrier()

    @pl.when(s == 0)
    def _():
        pltpu.sync_copy(dacc, deg_hbm.at[c])


_EXP = "splitgather"


def _edge_pipeline(g_hbm, acc, ridx_src, cidx_src, base, ibufs, bufs,
                   gsems, gsems2, ssems, irsems, icsems, nb):
    rbuf, cbuf = ibufs

    def iload(b, par, sync):
        rsrc = ridx_src.at[pl.ds(base + b * 8, 8)]
        csrc = cidx_src.at[pl.ds(base + b * 8, 8)]
        if sync:
            pltpu.sync_copy(rsrc, rbuf[par])
            pltpu.sync_copy(csrc, cbuf[par])
        else:
            pltpu.async_copy(rsrc, rbuf[par], irsems[par])
            pltpu.async_copy(csrc, cbuf[par], icsems[par])

    def iwait(par):
        pltpu.make_async_copy(ridx_src.at[pl.ds(0, 8)], rbuf[par], irsems[par]).wait()
        pltpu.make_async_copy(cidx_src.at[pl.ds(0, 8)], cbuf[par], icsems[par]).wait()

    def gwait(p):
        if _EXP == "splitgather":
            pltpu.make_async_copy(
                g_hbm.at[rbuf[0].at[0, pl.ds(0, 64)]],
                bufs[p].at[pl.ds(0, 64)], gsems[p],
            ).wait()
            pltpu.make_async_copy(
                g_hbm.at[rbuf[0].at[0, pl.ds(0, 64)]],
                bufs[p].at[pl.ds(64, 64)], gsems2[p],
            ).wait()
        else:
            pltpu.make_async_copy(g_hbm.at[rbuf[0].at[0]], bufs[p], gsems[p]).wait()

    def swait(p):
        if True:
            pltpu.make_async_copy(bufs[p], acc.at[cbuf[0].at[0]], ssems[p]).wait()

    def gissue(ib, j, p):
        if _EXP == "splitgather":
            pltpu.async_copy(
                g_hbm.at[rbuf[ib].at[j, pl.ds(0, 64)]],
                bufs[p].at[pl.ds(0, 64)], gsems[p],
            )
            pltpu.async_copy(
                g_hbm.at[rbuf[ib].at[j, pl.ds(64, 64)]],
                bufs[p].at[pl.ds(64, 64)], gsems2[p],
            )
        else:
            pltpu.async_copy(g_hbm.at[rbuf[ib].at[j]], bufs[p], gsems[p])

    def sissue(ib, j, p):
        if True:
            pltpu.async_copy(bufs[p], acc.at[cbuf[ib].at[j]], ssems[p], add=True)

    iload(0, 0, sync=True)
    gissue(0, 0, 0)

    def body(m, carry):
        for pb in range(2):
            b = 2 * m + pb
            for j in range(8):
                p = j % 2
                gwait(p)
                if j == 0:
                    @pl.when(b > 0)
                    def _():
                        swait(1)
                else:
                    swait(1 - p)
                if j < 7:
                    gissue(pb, j + 1, 1 - p)
                else:
                    @pl.when(b + 1 < nb)
                    def _():
                        iwait(1 - pb)
                        gissue(1 - pb, 0, 0)
                if j == 2:
                    @pl.when(b + 1 < nb)
                    def _():
                        iload(b + 1, 1 - pb, sync=False)
                sissue(pb, j, p)
        return carry

    lax.fori_loop(0, nb // 2, body, 0)
    swait(1)


def _make_agg_kernel(dh):
    @functools.partial(
        pl.kernel,
        out_type=jax.ShapeDtypeStruct((2 * N, dh), jnp.float32),
        mesh=_mesh,
        scratch_types=[
            pltpu.VMEM((8, 128), jnp.int32),
            pltpu.VMEM((8, 128), jnp.int32),
            pltpu.VMEM((8, 128), jnp.int32),
            pltpu.VMEM((8, 128), jnp.int32),
            pltpu.VMEM((128, dh), jnp.float32),
            pltpu.VMEM((128, dh), jnp.float32),
            pltpu.VMEM_SHARED((NP, dh), jnp.float32),
            pltpu.SemaphoreType.DMA,
            pltpu.SemaphoreType.DMA,
            pltpu.SemaphoreType.DMA,
            pltpu.SemaphoreType.DMA,
            pltpu.SemaphoreType.DMA,
            pltpu.SemaphoreType.DMA,
            pltpu.SemaphoreType.DMA,
            pltpu.SemaphoreType.DMA,
            pltpu.SemaphoreType.DMA,
            pltpu.SemaphoreType.DMA,
        ],
    )
    def agg(g_hbm, rowpair_hbm, col2_hbm, out_hbm, ridx0, ridx1, cidx0, cidx1,
            rows0, rows1, acc, gsem0, gsem1, ssem0, ssem1,
            irsem0, irsem1, icsem0, icsem1, g2sem0, g2sem1):
        c = lax.axis_index("c")
        s = lax.axis_index("s")

        @pl.when(s < 15)
        def _():
            pltpu.sync_copy(
                g_hbm.at[pl.ds(c * N + s * 640, 640)], acc.at[pl.ds(s * 640, 640)]
            )

        @pl.when(s == 15)
        def _():
            pltpu.sync_copy(
                g_hbm.at[pl.ds(c * N + 9600, 400)], acc.at[pl.ds(9600, 400)]
            )

        plsc.subcore_barrier()

        _edge_pipeline(
            g_hbm, acc, rowpair_hbm.at[c], col2_hbm, s * RPT,
            ((ridx0, ridx1), (cidx0, cidx1)), (rows0, rows1),
            (gsem0, gsem1), (g2sem0, g2sem1), (ssem0, ssem1),
            (irsem0, irsem1), (icsem0, icsem1), RPT // 8,
        )
        plsc.subcore_barrier()

        @pl.when(s < 15)
        def _():
            pltpu.sync_copy(
                acc.at[pl.ds(s * 640, 640)], out_hbm.at[pl.ds(c * N + s * 640, 640)]
            )

        @pl.when(s == 15)
        def _():
            pltpu.sync_copy(
                acc.at[pl.ds(9600, 400)], out_hbm.at[pl.ds(c * N + 9600, 400)]
            )

    return agg


_agg1 = _make_agg_kernel(D_HID // 2)


@functools.partial(
    pl.kernel,
    out_type=jax.ShapeDtypeStruct((NC, N, D_OUT), jnp.float32),
    mesh=_mesh,
    scratch_types=[
        pltpu.VMEM((8, 128), jnp.int32),
        pltpu.VMEM((8, 128), jnp.int32),
        pltpu.VMEM((8, 128), jnp.int32),
        pltpu.VMEM((8, 128), jnp.int32),
        pltpu.VMEM((128, D_OUT), jnp.float32),
        pltpu.VMEM((128, D_OUT), jnp.float32),
        pltpu.VMEM_SHARED((NP, D_OUT), jnp.float32),
        pltpu.SemaphoreType.DMA,
        pltpu.SemaphoreType.DMA,
        pltpu.SemaphoreType.DMA,
        pltpu.SemaphoreType.DMA,
        pltpu.SemaphoreType.DMA,
        pltpu.SemaphoreType.DMA,
        pltpu.SemaphoreType.DMA,
        pltpu.SemaphoreType.DMA,
        pltpu.SemaphoreType.DMA,
        pltpu.SemaphoreType.DMA,
    ],
)
def _agg2(g_hbm, row2_hbm, col2_hbm, out_hbm, ridx0, ridx1, cidx0, cidx1,
          rows0, rows1, acc, gsem0, gsem1, ssem0, ssem1,
          irsem0, irsem1, icsem0, icsem1, g2sem0, g2sem1):
    c = lax.axis_index("c")
    s = lax.axis_index("s")

    @pl.when(s < 15)
    def _():
        pltpu.sync_copy(g_hbm.at[pl.ds(s * 640, 640)], acc.at[pl.ds(s * 640, 640)])

    @pl.when(s == 15)
    def _():
        pltpu.sync_copy(g_hbm.at[pl.ds(9600, 400)], acc.at[pl.ds(9600, 400)])

    rpt = ER // (NC * NS)
    plsc.subcore_barrier()

    _edge_pipeline(
        g_hbm, acc, row2_hbm, col2_hbm, (s * NC + c) * rpt,
        ((ridx0, ridx1), (cidx0, cidx1)), (rows0, rows1),
        (gsem0, gsem1), (g2sem0, g2sem1), (ssem0, ssem1),
        (irsem0, irsem1), (icsem0, icsem1), rpt // 8,
    )
    plsc.subcore_barrier()

    @pl.when(s < 15)
    def _():
        pltpu.sync_copy(acc.at[pl.ds(s * 640, 640)], out_hbm.at[c, pl.ds(s * 640, 640)])

    @pl.when(s == 15)
    def _():
        pltpu.sync_copy(acc.at[pl.ds(9600, 400)], out_hbm.at[c, pl.ds(9600, 400)])

_B = 1000
_NB = N // _B


def _tc0_body(x_ref, w1_ref, h1_ref):
    h1_ref[...] = jnp.dot(
        x_ref[...], w1_ref[...], preferred_element_type=jnp.float32
    )


def _tc1_body(pdeg_ref, h1_ref, dis_ref, g1_ref):
    deg = pdeg_ref[0] + pdeg_ref[1] - 1.0
    dis = lax.rsqrt(deg)
    dis_ref[...] = dis
    g = h1_ref[...] * dis
    g1_ref[0] = g[:, : D_HID // 2]
    g1_ref[1] = g[:, D_HID // 2 :]


def _tc2_body(agg1_ref, dis_ref, b1_ref, w2_ref, g2_ref):
    dis = dis_ref[...]
    o_l = jnp.maximum(agg1_ref[0] * dis + b1_ref[0, : D_HID // 2], 0.0)
    o_r = jnp.maximum(agg1_ref[1] * dis + b1_ref[0, D_HID // 2 :], 0.0)
    h2 = jnp.dot(o_l, w2_ref[: D_HID // 2], preferred_element_type=jnp.float32)
    h2 = h2 + jnp.dot(o_r, w2_ref[D_HID // 2 :], preferred_element_type=jnp.float32)
    g2_ref[...] = h2 * dis


def _tc3_body(agg2_ref, g2_ref, dis_ref, b2_ref, out_ref):
    full = agg2_ref[0] + agg2_ref[1] - g2_ref[...]
    out_ref[...] = full * dis_ref[...] + b2_ref[...]


_tc0 = pl.pallas_call(
    _tc0_body,
    grid=(_NB,),
    in_specs=[
        pl.BlockSpec((_B, D_IN), lambda i: (i, 0)),
        pl.BlockSpec((D_IN, D_HID), lambda i: (0, 0)),
    ],
    out_specs=pl.BlockSpec((_B, D_HID), lambda i: (i, 0)),
    out_shape=jax.ShapeDtypeStruct((N, D_HID), jnp.float32),
)

_tc1 = pl.pallas_call(
    _tc1_body,
    grid=(_NB,),
    in_specs=[
        pl.BlockSpec((2, _B, 1), lambda i: (0, i, 0)),
        pl.BlockSpec((_B, D_HID), lambda i: (i, 0)),
    ],
    out_specs=[
        pl.BlockSpec((_B, 1), lambda i: (i, 0)),
        pl.BlockSpec((2, _B, D_HID // 2), lambda i: (0, i, 0)),
    ],
    out_shape=[
        jax.ShapeDtypeStruct((N, 1), jnp.float32),
        jax.ShapeDtypeStruct((2, N, D_HID // 2), jnp.float32),
    ],
)

_tc2 = pl.pallas_call(
    _tc2_body,
    grid=(_NB,),
    in_specs=[
        pl.BlockSpec((2, _B, D_HID // 2), lambda i: (0, i, 0)),
        pl.BlockSpec((_B, 1), lambda i: (i, 0)),
        pl.BlockSpec((1, D_HID), lambda i: (0, 0)),
        pl.BlockSpec((D_HID, D_OUT), lambda i: (0, 0)),
    ],
    out_specs=pl.BlockSpec((_B, D_OUT), lambda i: (i, 0)),
    out_shape=jax.ShapeDtypeStruct((N, D_OUT), jnp.float32),
)

_tc3 = pl.pallas_call(
    _tc3_body,
    grid=(_NB,),
    in_specs=[
        pl.BlockSpec((2, _B, D_OUT), lambda i: (0, i, 0)),
        pl.BlockSpec((_B, D_OUT), lambda i: (i, 0)),
        pl.BlockSpec((_B, 1), lambda i: (i, 0)),
        pl.BlockSpec((1, D_OUT), lambda i: (0, 0)),
    ],
    out_specs=pl.BlockSpec((_B, D_OUT), lambda i: (i, 0)),
    out_shape=jax.ShapeDtypeStruct((N, D_OUT), jnp.float32),
)


def kernel(x, edge_index, W1, b1, W2, b2):
    row = edge_index[0]
    col = edge_index[1]
    pad_i = jnp.arange(PAD, dtype=jnp.int32)
    rowp = jnp.concatenate([row, pad_i % 64])
    colp = jnp.concatenate([col, N + (pad_i % 16)])
    col2 = colp.reshape(ER, 128)
    row2 = rowp.reshape(ER, 128)
    rowpair = jnp.stack([rowp, rowp + N]).reshape(2, ER, 128)
    ones = jnp.ones((NP,), jnp.float32)

    h1 = _tc0(x, W1)
    deg2 = _deg_kernel(ones, col2)
    pdeg = deg2[:, :N].reshape(2, N, 1)
    dis, g1 = _tc1(pdeg, h1)
    agg1 = _agg1(g1.reshape(2 * N, D_HID // 2), rowpair, col2)
    g2 = _tc2(agg1.reshape(2, N, D_HID // 2), dis, b1.reshape(1, D_HID), W2)
    agg2 = _agg2(g2, row2, col2)
    out = _tc3(agg2, g2, dis, b2.reshape(1, D_OUT))
    return out


# speedup vs baseline: 1.0260x; 1.0014x over previous
import functools

import jax
import jax.numpy as jnp
from jax import lax
from jax.experimental import pallas as pl
from jax.experimental.pallas import tpu as pltpu
from jax.experimental.pallas import tpu_sc as plsc

N = 10000
E = 320000
D_IN = 128
D_HID = 256
D_OUT = 128

NC = 2
NS = 16

EP = 327680
ER = EP // 128
PAD = EP - E
NP = N + 16
RPT = ER // NS
RPW = ER // (NC * NS)

_mesh = plsc.VectorSubcoreMesh(core_axis_name="c", subcore_axis_name="s")

@functools.partial(
    pl.kernel,
    out_type=jax.ShapeDtypeStruct((NC, NP), jnp.float32),
    mesh=_mesh,
    scratch_types=[
        pltpu.VMEM((8, 128), jnp.int32),
        pltpu.VMEM((128,), jnp.float32),
        pltpu.VMEM_SHARED((NP,), jnp.float32),
    ],
)
def _deg_kernel(ones_hbm, col2_hbm, deg_hbm, cidx, ones_v, dacc):
    c = lax.axis_index("c")
    s = lax.axis_index("s")
    w = s * NC + c

    @pl.when(s == 0)
    def _():
        pltpu.sync_copy(ones_hbm, dacc)

    pltpu.sync_copy(ones_hbm.at[pl.ds(0, 128)], ones_v)
    plsc.subcore_barrier()

    def body(o, carry):
        rb = w * RPW + o * 8
        pltpu.sync_copy(col2_hbm.at[pl.ds(rb, 8)], cidx)
        for j in range(8):
            pltpu.sync_copy(ones_v, dacc.at[cidx.at[j]], add=True)
        return carry

    lax.fori_loop(0, RPW // 8, body, 0)
    plsc.subcore_barrier()

    @pl.when(s == 0)
    def _():
        pltpu.sync_copy(dacc, deg_hbm.at[c])


def _edge_pipeline(g_hbm, acc, ridx_src, cidx_src, base, ibufs, bufs,
                   gsems, gsems2, ssems, irsems, icsems, nb):
    rbuf, cbuf = ibufs

    def iload(b, par, sync):
        rsrc = ridx_src.at[pl.ds(base + b * 8, 8)]
        csrc = cidx_src.at[pl.ds(base + b * 8, 8)]
        if sync:
            pltpu.sync_copy(rsrc, rbuf[par])
            pltpu.sync_copy(csrc, cbuf[par])
        else:
            pltpu.async_copy(rsrc, rbuf[par], irsems[par])
            pltpu.async_copy(csrc, cbuf[par], icsems[par])

    def iwait(par):
        pltpu.make_async_copy(ridx_src.at[pl.ds(0, 8)], rbuf[par], irsems[par]).wait()
        pltpu.make_async_copy(cidx_src.at[pl.ds(0, 8)], cbuf[par], icsems[par]).wait()

    def gwait(p):
        pltpu.make_async_copy(
                g_hbm.at[rbuf[0].at[0, pl.ds(0, 64)]],
                bufs[p].at[pl.ds(0, 64)], gsems[p],
        ).wait()
        pltpu.make_async_copy(
                g_hbm.at[rbuf[0].at[0, pl.ds(0, 64)]],
                bufs[p].at[pl.ds(64, 64)], gsems2[p],
        ).wait()

    def swait(p):
        pltpu.make_async_copy(bufs[p], acc.at[cbuf[0].at[0]], ssems[p]).wait()

    def gissue(ib, j, p):
        pltpu.async_copy(
                g_hbm.at[rbuf[ib].at[j, pl.ds(0, 64)]],
                bufs[p].at[pl.ds(0, 64)], gsems[p],
        )
        pltpu.async_copy(
                g_hbm.at[rbuf[ib].at[j, pl.ds(64, 64)]],
                bufs[p].at[pl.ds(64, 64)], gsems2[p],
        )

    def sissue(ib, j, p):
        pltpu.async_copy(bufs[p], acc.at[cbuf[ib].at[j]], ssems[p], add=True)

    iload(0, 0, sync=True)
    gissue(0, 0, 0)

    def body(m, carry):
        for pb in range(2):
            b = 2 * m + pb
            for j in range(8):
                p = j % 2
                gwait(p)
                if j == 0:
                    @pl.when(b > 0)
                    def _():
                        swait(1)
                else:
                    swait(1 - p)
                if j < 7:
                    gissue(pb, j + 1, 1 - p)
                else:
                    @pl.when(b + 1 < nb)
                    def _():
                        iwait(1 - pb)
                        gissue(1 - pb, 0, 0)
                if j == 2:
                    @pl.when(b + 1 < nb)
                    def _():
                        iload(b + 1, 1 - pb, sync=False)
                sissue(pb, j, p)
        return carry

    lax.fori_loop(0, nb // 2, body, 0)
    swait(1)


def _make_agg_kernel(dh):
    @functools.partial(
        pl.kernel,
        out_type=jax.ShapeDtypeStruct((2 * N, dh), jnp.float32),
        mesh=_mesh,
        scratch_types=[
            pltpu.VMEM((8, 128), jnp.int32),
            pltpu.VMEM((8, 128), jnp.int32),
            pltpu.VMEM((8, 128), jnp.int32),
            pltpu.VMEM((8, 128), jnp.int32),
            pltpu.VMEM((128, dh), jnp.float32),
            pltpu.VMEM((128, dh), jnp.float32),
            pltpu.VMEM_SHARED((NP, dh), jnp.float32),
            pltpu.SemaphoreType.DMA,
            pltpu.SemaphoreType.DMA,
            pltpu.SemaphoreType.DMA,
            pltpu.SemaphoreType.DMA,
            pltpu.SemaphoreType.DMA,
            pltpu.SemaphoreType.DMA,
            pltpu.SemaphoreType.DMA,
            pltpu.SemaphoreType.DMA,
            pltpu.SemaphoreType.DMA,
            pltpu.SemaphoreType.DMA,
        ],
    )
    def agg(g_hbm, rowpair_hbm, col2_hbm, out_hbm, ridx0, ridx1, cidx0, cidx1,
            rows0, rows1, acc, gsem0, gsem1, ssem0, ssem1,
            irsem0, irsem1, icsem0, icsem1, g2sem0, g2sem1):
        c = lax.axis_index("c")
        s = lax.axis_index("s")

        @pl.when(s < 15)
        def _():
            pltpu.sync_copy(
                g_hbm.at[pl.ds(c * N + s * 640, 640)], acc.at[pl.ds(s * 640, 640)]
            )

        @pl.when(s == 15)
        def _():
            pltpu.sync_copy(
                g_hbm.at[pl.ds(c * N + 9600, 400)], acc.at[pl.ds(9600, 400)]
            )

        plsc.subcore_barrier()

        _edge_pipeline(
            g_hbm, acc, rowpair_hbm.at[c], col2_hbm, s * RPT,
            ((ridx0, ridx1), (cidx0, cidx1)), (rows0, rows1),
            (gsem0, gsem1), (g2sem0, g2sem1), (ssem0, ssem1),
            (irsem0, irsem1), (icsem0, icsem1), RPT // 8,
        )
        plsc.subcore_barrier()

        @pl.when(s < 15)
        def _():
            pltpu.sync_copy(
                acc.at[pl.ds(s * 640, 640)], out_hbm.at[pl.ds(c * N + s * 640, 640)]
            )

        @pl.when(s == 15)
        def _():
            pltpu.sync_copy(
                acc.at[pl.ds(9600, 400)], out_hbm.at[pl.ds(c * N + 9600, 400)]
            )

    return agg


_agg1 = _make_agg_kernel(D_HID // 2)


@functools.partial(
    pl.kernel,
    out_type=jax.ShapeDtypeStruct((NC, N, D_OUT), jnp.float32),
    mesh=_mesh,
    scratch_types=[
        pltpu.VMEM((8, 128), jnp.int32),
        pltpu.VMEM((8, 128), jnp.int32),
        pltpu.VMEM((8, 128), jnp.int32),
        pltpu.VMEM((8, 128), jnp.int32),
        pltpu.VMEM((128, D_OUT), jnp.float32),
        pltpu.VMEM((128, D_OUT), jnp.float32),
        pltpu.VMEM_SHARED((NP, D_OUT), jnp.float32),
        pltpu.SemaphoreType.DMA,
        pltpu.SemaphoreType.DMA,
        pltpu.SemaphoreType.DMA,
        pltpu.SemaphoreType.DMA,
        pltpu.SemaphoreType.DMA,
        pltpu.SemaphoreType.DMA,
        pltpu.SemaphoreType.DMA,
        pltpu.SemaphoreType.DMA,
        pltpu.SemaphoreType.DMA,
        pltpu.SemaphoreType.DMA,
    ],
)
def _agg2(g_hbm, row2_hbm, col2_hbm, out_hbm, ridx0, ridx1, cidx0, cidx1,
          rows0, rows1, acc, gsem0, gsem1, ssem0, ssem1,
          irsem0, irsem1, icsem0, icsem1, g2sem0, g2sem1):
    c = lax.axis_index("c")
    s = lax.axis_index("s")

    @pl.when(s < 15)
    def _():
        pltpu.sync_copy(g_hbm.at[pl.ds(s * 640, 640)], acc.at[pl.ds(s * 640, 640)])

    @pl.when(s == 15)
    def _():
        pltpu.sync_copy(g_hbm.at[pl.ds(9600, 400)], acc.at[pl.ds(9600, 400)])

    rpt = ER // (NC * NS)
    plsc.subcore_barrier()

    _edge_pipeline(
        g_hbm, acc, row2_hbm, col2_hbm, (s * NC + c) * rpt,
        ((ridx0, ridx1), (cidx0, cidx1)), (rows0, rows1),
        (gsem0, gsem1), (g2sem0, g2sem1), (ssem0, ssem1),
        (irsem0, irsem1), (icsem0, icsem1), rpt // 8,
    )
    plsc.subcore_barrier()

    @pl.when(s < 15)
    def _():
        pltpu.sync_copy(acc.at[pl.ds(s * 640, 640)], out_hbm.at[c, pl.ds(s * 640, 640)])

    @pl.when(s == 15)
    def _():
        pltpu.sync_copy(acc.at[pl.ds(9600, 400)], out_hbm.at[c, pl.ds(9600, 400)])

_B = 1000
_NB = N // _B


def _tc0_body(x_ref, w1_ref, h1_ref):
    h1_ref[...] = jnp.dot(
        x_ref[...], w1_ref[...], preferred_element_type=jnp.float32
    )


def _tc1_body(pdeg_ref, h1_ref, dis_ref, g1_ref):
    deg = pdeg_ref[0] + pdeg_ref[1] - 1.0
    dis = lax.rsqrt(deg)
    dis_ref[...] = dis
    g = h1_ref[...] * dis
    g1_ref[0] = g[:, : D_HID // 2]
    g1_ref[1] = g[:, D_HID // 2 :]


def _tc2_body(agg1_ref, dis_ref, b1_ref, w2_ref, g2_ref):
    dis = dis_ref[...]
    o_l = jnp.maximum(agg1_ref[0] * dis + b1_ref[0, : D_HID // 2], 0.0)
    o_r = jnp.maximum(agg1_ref[1] * dis + b1_ref[0, D_HID // 2 :], 0.0)
    h2 = jnp.dot(o_l, w2_ref[: D_HID // 2], preferred_element_type=jnp.float32)
    h2 = h2 + jnp.dot(o_r, w2_ref[D_HID // 2 :], preferred_element_type=jnp.float32)
    g2_ref[...] = h2 * dis


def _tc3_body(agg2_ref, g2_ref, dis_ref, b2_ref, out_ref):
    full = agg2_ref[0] + agg2_ref[1] - g2_ref[...]
    out_ref[...] = full * dis_ref[...] + b2_ref[...]


_tc0 = pl.pallas_call(
    _tc0_body,
    grid=(_NB,),
    in_specs=[
        pl.BlockSpec((_B, D_IN), lambda i: (i, 0)),
        pl.BlockSpec((D_IN, D_HID), lambda i: (0, 0)),
    ],
    out_specs=pl.BlockSpec((_B, D_HID), lambda i: (i, 0)),
    out_shape=jax.ShapeDtypeStruct((N, D_HID), jnp.float32),
)

_tc1 = pl.pallas_call(
    _tc1_body,
    grid=(_NB,),
    in_specs=[
        pl.BlockSpec((2, _B, 1), lambda i: (0, i, 0)),
        pl.BlockSpec((_B, D_HID), lambda i: (i, 0)),
    ],
    out_specs=[
        pl.BlockSpec((_B, 1), lambda i: (i, 0)),
        pl.BlockSpec((2, _B, D_HID // 2), lambda i: (0, i, 0)),
    ],
    out_shape=[
        jax.ShapeDtypeStruct((N, 1), jnp.float32),
        jax.ShapeDtypeStruct((2, N, D_HID // 2), jnp.float32),
    ],
)

_tc2 = pl.pallas_call(
    _tc2_body,
    grid=(_NB,),
    in_specs=[
        pl.BlockSpec((2, _B, D_HID // 2), lambda i: (0, i, 0)),
        pl.BlockSpec((_B, 1), lambda i: (i, 0)),
        pl.BlockSpec((1, D_HID), lambda i: (0, 0)),
        pl.BlockSpec((D_HID, D_OUT), lambda i: (0, 0)),
    ],
    out_specs=pl.BlockSpec((_B, D_OUT), lambda i: (i, 0)),
    out_shape=jax.ShapeDtypeStruct((N, D_OUT), jnp.float32),
)

_tc3 = pl.pallas_call(
    _tc3_body,
    grid=(_NB,),
    in_specs=[
        pl.BlockSpec((2, _B, D_OUT), lambda i: (0, i, 0)),
        pl.BlockSpec((_B, D_OUT), lambda i: (i, 0)),
        pl.BlockSpec((_B, 1), lambda i: (i, 0)),
        pl.BlockSpec((1, D_OUT), lambda i: (0, 0)),
    ],
    out_specs=pl.BlockSpec((_B, D_OUT), lambda i: (i, 0)),
    out_shape=jax.ShapeDtypeStruct((N, D_OUT), jnp.float32),
)


def kernel(x, edge_index, W1, b1, W2, b2):
    row = edge_index[0]
    col = edge_index[1]
    pad_i = jnp.arange(PAD, dtype=jnp.int32)
    rowp = jnp.concatenate([row, pad_i % N])
    colp = jnp.concatenate([col, N + (pad_i % 16)])
    col2 = colp.reshape(ER, 128)
    row2 = rowp.reshape(ER, 128)
    rowpair = jnp.stack([rowp, rowp + N]).reshape(2, ER, 128)
    ones = jnp.ones((NP,), jnp.float32)

    h1 = _tc0(x, W1)
    deg2 = _deg_kernel(ones, col2)
    pdeg = deg2[:, :N].reshape(2, N, 1)
    dis, g1 = _tc1(pdeg, h1)
    agg1 = _agg1(g1.reshape(2 * N, D_HID // 2), rowpair, col2)
    g2 = _tc2(agg1.reshape(2, N, D_HID // 2), dis, b1.reshape(1, D_HID), W2)
    agg2 = _agg2(g2, row2, col2)
    out = _tc3(agg2, g2, dis, b2.reshape(1, D_OUT))
    return out


# speedup vs baseline: 1.0596x; 1.0327x over previous
import functools

import jax
import jax.numpy as jnp
from jax import lax
from jax.experimental import pallas as pl
from jax.experimental.pallas import tpu as pltpu
from jax.experimental.pallas import tpu_sc as plsc

N = 10000
E = 320000
D_IN = 128
D_HID = 256
D_OUT = 128

NC = 2
NS = 16

EP = 327680
ER = EP // 128
PAD = EP - E
NP = N + 16
RPT = ER // NS
RPW = ER // (NC * NS)

_mesh = plsc.VectorSubcoreMesh(core_axis_name="c", subcore_axis_name="s")

@functools.partial(
    pl.kernel,
    out_type=jax.ShapeDtypeStruct((NC, NP), jnp.float32),
    mesh=_mesh,
    scratch_types=[
        pltpu.VMEM((8, 128), jnp.int32),
        pltpu.VMEM((128,), jnp.float32),
        pltpu.VMEM_SHARED((NP,), jnp.float32),
    ],
)
def _deg_kernel(ones_hbm, col2_hbm, deg_hbm, cidx, ones_v, dacc):
    c = lax.axis_index("c")
    s = lax.axis_index("s")
    w = s * NC + c

    @pl.when(s == 0)
    def _():
        pltpu.sync_copy(ones_hbm, dacc)

    pltpu.sync_copy(ones_hbm.at[pl.ds(0, 128)], ones_v)
    plsc.subcore_barrier()

    def body(o, carry):
        rb = w * RPW + o * 8
        pltpu.sync_copy(col2_hbm.at[pl.ds(rb, 8)], cidx)
        for j in range(8):
            pltpu.sync_copy(ones_v, dacc.at[cidx.at[j]], add=True)
        return carry

    lax.fori_loop(0, RPW // 8, body, 0)
    plsc.subcore_barrier()

    @pl.when(s == 0)
    def _():
        pltpu.sync_copy(dacc, deg_hbm.at[c])


def _edge_pipeline(g_hbm, acc, ridx_src, cidx_src, base, ibufs, bufs,
                   gsems, ssems, irsems, icsems, nb):
    rbuf, cbuf = ibufs

    def iload(b, par, sync):
        rsrc = ridx_src.at[pl.ds(base + b * 8, 8)]
        csrc = cidx_src.at[pl.ds(base + b * 8, 8)]
        if sync:
            pltpu.sync_copy(rsrc, rbuf[par])
            pltpu.sync_copy(csrc, cbuf[par])
        else:
            pltpu.async_copy(rsrc, rbuf[par], irsems[par])
            pltpu.async_copy(csrc, cbuf[par], icsems[par])

    def iwait(par):
        pltpu.make_async_copy(ridx_src.at[pl.ds(0, 8)], rbuf[par], irsems[par]).wait()
        pltpu.make_async_copy(cidx_src.at[pl.ds(0, 8)], cbuf[par], icsems[par]).wait()

    def gwait(p):
        pltpu.make_async_copy(
            g_hbm.at[rbuf[0].at[0, pl.ds(0, 64)]], bufs[p], gsems[p]
        ).wait()

    def swait(p):
        pltpu.make_async_copy(
            bufs[p], acc.at[cbuf[0].at[0, pl.ds(0, 64)]], ssems[p]
        ).wait()

    def gissue(ib, l, p):
        pltpu.async_copy(
            g_hbm.at[rbuf[ib].at[l // 2, pl.ds(64 * (l % 2), 64)]], bufs[p], gsems[p]
        )

    def sissue(ib, l, p):
        pltpu.async_copy(
            bufs[p], acc.at[cbuf[ib].at[l // 2, pl.ds(64 * (l % 2), 64)]],
            ssems[p], add=True,
        )

    iload(0, 0, sync=True)
    gissue(0, 0, 0)
    gissue(0, 1, 1)

    def body(m, carry):
        for pb in range(2):
            b = 2 * m + pb
            for l in range(16):
                p = l % 4
                gwait(p)
                if l >= 2:
                    swait((l - 2) % 4)
                else:
                    @pl.when(b > 0)
                    def _():
                        swait((l - 2) % 4)
                if l < 14:
                    gissue(pb, l + 2, (l + 2) % 4)
                else:
                    @pl.when(b + 1 < nb)
                    def _():
                        if l == 14:
                            iwait(1 - pb)
                        gissue(1 - pb, l - 14, (l + 2) % 4)
                if l == 4:
                    @pl.when(b + 1 < nb)
                    def _():
                        iload(b + 1, 1 - pb, sync=False)
                sissue(pb, l, p)
        return carry

    lax.fori_loop(0, nb // 2, body, 0)
    swait(2)
    swait(3)


def _make_agg_kernel(dh):
    @functools.partial(
        pl.kernel,
        out_type=jax.ShapeDtypeStruct((2 * N, dh), jnp.float32),
        mesh=_mesh,
        scratch_types=[
            pltpu.VMEM((8, 128), jnp.int32),
            pltpu.VMEM((8, 128), jnp.int32),
            pltpu.VMEM((8, 128), jnp.int32),
            pltpu.VMEM((8, 128), jnp.int32),
            pltpu.VMEM((64, dh), jnp.float32),
            pltpu.VMEM((64, dh), jnp.float32),
            pltpu.VMEM((64, dh), jnp.float32),
            pltpu.VMEM((64, dh), jnp.float32),
            pltpu.VMEM_SHARED((NP, dh), jnp.float32),
            pltpu.SemaphoreType.DMA,
            pltpu.SemaphoreType.DMA,
            pltpu.SemaphoreType.DMA,
            pltpu.SemaphoreType.DMA,
            pltpu.SemaphoreType.DMA,
            pltpu.SemaphoreType.DMA,
            pltpu.SemaphoreType.DMA,
            pltpu.SemaphoreType.DMA,
            pltpu.SemaphoreType.DMA,
            pltpu.SemaphoreType.DMA,
            pltpu.SemaphoreType.DMA,
            pltpu.SemaphoreType.DMA,
        ],
    )
    def agg(g_hbm, rowpair_hbm, col2_hbm, out_hbm, ridx0, ridx1, cidx0, cidx1,
            rows0, rows1, rows2, rows3, acc,
            gsem0, gsem1, gsem2, gsem3, ssem0, ssem1, ssem2, ssem3,
            irsem0, irsem1, icsem0, icsem1):
        c = lax.axis_index("c")
        s = lax.axis_index("s")

        @pl.when(s < 15)
        def _():
            pltpu.sync_copy(
                g_hbm.at[pl.ds(c * N + s * 640, 640)], acc.at[pl.ds(s * 640, 640)]
            )

        @pl.when(s == 15)
        def _():
            pltpu.sync_copy(
                g_hbm.at[pl.ds(c * N + 9600, 400)], acc.at[pl.ds(9600, 400)]
            )

        plsc.subcore_barrier()

        _edge_pipeline(
            g_hbm, acc, rowpair_hbm.at[c], col2_hbm, s * RPT,
            ((ridx0, ridx1), (cidx0, cidx1)), (rows0, rows1, rows2, rows3),
            (gsem0, gsem1, gsem2, gsem3), (ssem0, ssem1, ssem2, ssem3),
            (irsem0, irsem1), (icsem0, icsem1), RPT // 8,
        )
        plsc.subcore_barrier()

        @pl.when(s < 15)
        def _():
            pltpu.sync_copy(
                acc.at[pl.ds(s * 640, 640)], out_hbm.at[pl.ds(c * N + s * 640, 640)]
            )

        @pl.when(s == 15)
        def _():
            pltpu.sync_copy(
                acc.at[pl.ds(9600, 400)], out_hbm.at[pl.ds(c * N + 9600, 400)]
            )

    return agg


_agg1 = _make_agg_kernel(D_HID // 2)


@functools.partial(
    pl.kernel,
    out_type=jax.ShapeDtypeStruct((NC, N, D_OUT), jnp.float32),
    mesh=_mesh,
    scratch_types=[
        pltpu.VMEM((8, 128), jnp.int32),
        pltpu.VMEM((8, 128), jnp.int32),
        pltpu.VMEM((8, 128), jnp.int32),
        pltpu.VMEM((8, 128), jnp.int32),
        pltpu.VMEM((64, D_OUT), jnp.float32),
        pltpu.VMEM((64, D_OUT), jnp.float32),
        pltpu.VMEM((64, D_OUT), jnp.float32),
        pltpu.VMEM((64, D_OUT), jnp.float32),
        pltpu.VMEM_SHARED((NP, D_OUT), jnp.float32),
        pltpu.SemaphoreType.DMA,
        pltpu.SemaphoreType.DMA,
        pltpu.SemaphoreType.DMA,
        pltpu.SemaphoreType.DMA,
        pltpu.SemaphoreType.DMA,
        pltpu.SemaphoreType.DMA,
        pltpu.SemaphoreType.DMA,
        pltpu.SemaphoreType.DMA,
        pltpu.SemaphoreType.DMA,
        pltpu.SemaphoreType.DMA,
        pltpu.SemaphoreType.DMA,
        pltpu.SemaphoreType.DMA,
    ],
)
def _agg2(g_hbm, row2_hbm, col2_hbm, out_hbm, ridx0, ridx1, cidx0, cidx1,
          rows0, rows1, rows2, rows3, acc,
          gsem0, gsem1, gsem2, gsem3, ssem0, ssem1, ssem2, ssem3,
          irsem0, irsem1, icsem0, icsem1):
    c = lax.axis_index("c")
    s = lax.axis_index("s")

    @pl.when(s < 15)
    def _():
        pltpu.sync_copy(g_hbm.at[pl.ds(s * 640, 640)], acc.at[pl.ds(s * 640, 640)])

    @pl.when(s == 15)
    def _():
        pltpu.sync_copy(g_hbm.at[pl.ds(9600, 400)], acc.at[pl.ds(9600, 400)])

    rpt = ER // (NC * NS)
    plsc.subcore_barrier()

    _edge_pipeline(
        g_hbm, acc, row2_hbm, col2_hbm, (s * NC + c) * rpt,
        ((ridx0, ridx1), (cidx0, cidx1)), (rows0, rows1, rows2, rows3),
        (gsem0, gsem1, gsem2, gsem3), (ssem0, ssem1, ssem2, ssem3),
        (irsem0, irsem1), (icsem0, icsem1), rpt // 8,
    )
    plsc.subcore_barrier()

    @pl.when(s < 15)
    def _():
        pltpu.sync_copy(acc.at[pl.ds(s * 640, 640)], out_hbm.at[c, pl.ds(s * 640, 640)])

    @pl.when(s == 15)
    def _():
        pltpu.sync_copy(acc.at[pl.ds(9600, 400)], out_hbm.at[c, pl.ds(9600, 400)])

_B = 1000
_NB = N // _B


def _tc0_body(x_ref, w1_ref, h1_ref):
    h1_ref[...] = jnp.dot(
        x_ref[...], w1_ref[...], preferred_element_type=jnp.float32
    )


def _tc1_body(pdeg_ref, h1_ref, dis_ref, g1_ref):
    deg = pdeg_ref[0] + pdeg_ref[1] - 1.0
    dis = lax.rsqrt(deg)
    dis_ref[...] = dis
    g = h1_ref[...] * dis
    g1_ref[0] = g[:, : D_HID // 2]
    g1_ref[1] = g[:, D_HID // 2 :]


def _tc2_body(agg1_ref, dis_ref, b1_ref, w2_ref, g2_ref):
    dis = dis_ref[...]
    o_l = jnp.maximum(agg1_ref[0] * dis + b1_ref[0, : D_HID // 2], 0.0)
    o_r = jnp.maximum(agg1_ref[1] * dis + b1_ref[0, D_HID // 2 :], 0.0)
    h2 = jnp.dot(o_l, w2_ref[: D_HID // 2], preferred_element_type=jnp.float32)
    h2 = h2 + jnp.dot(o_r, w2_ref[D_HID // 2 :], preferred_element_type=jnp.float32)
    g2_ref[...] = h2 * dis


def _tc3_body(agg2_ref, g2_ref, dis_ref, b2_ref, out_ref):
    full = agg2_ref[0] + agg2_ref[1] - g2_ref[...]
    out_ref[...] = full * dis_ref[...] + b2_ref[...]


_tc0 = pl.pallas_call(
    _tc0_body,
    grid=(_NB,),
    in_specs=[
        pl.BlockSpec((_B, D_IN), lambda i: (i, 0)),
        pl.BlockSpec((D_IN, D_HID), lambda i: (0, 0)),
    ],
    out_specs=pl.BlockSpec((_B, D_HID), lambda i: (i, 0)),
    out_shape=jax.ShapeDtypeStruct((N, D_HID), jnp.float32),
)

_tc1 = pl.pallas_call(
    _tc1_body,
    grid=(_NB,),
    in_specs=[
        pl.BlockSpec((2, _B, 1), lambda i: (0, i, 0)),
        pl.BlockSpec((_B, D_HID), lambda i: (i, 0)),
    ],
    out_specs=[
        pl.BlockSpec((_B, 1), lambda i: (i, 0)),
        pl.BlockSpec((2, _B, D_HID // 2), lambda i: (0, i, 0)),
    ],
    out_shape=[
        jax.ShapeDtypeStruct((N, 1), jnp.float32),
        jax.ShapeDtypeStruct((2, N, D_HID // 2), jnp.float32),
    ],
)

_tc2 = pl.pallas_call(
    _tc2_body,
    grid=(_NB,),
    in_specs=[
        pl.BlockSpec((2, _B, D_HID // 2), lambda i: (0, i, 0)),
        pl.BlockSpec((_B, 1), lambda i: (i, 0)),
        pl.BlockSpec((1, D_HID), lambda i: (0, 0)),
        pl.BlockSpec((D_HID, D_OUT), lambda i: (0, 0)),
    ],
    out_specs=pl.BlockSpec((_B, D_OUT), lambda i: (i, 0)),
    out_shape=jax.ShapeDtypeStruct((N, D_OUT), jnp.float32),
)

_tc3 = pl.pallas_call(
    _tc3_body,
    grid=(_NB,),
    in_specs=[
        pl.BlockSpec((2, _B, D_OUT), lambda i: (0, i, 0)),
        pl.BlockSpec((_B, D_OUT), lambda i: (i, 0)),
        pl.BlockSpec((_B, 1), lambda i: (i, 0)),
        pl.BlockSpec((1, D_OUT), lambda i: (0, 0)),
    ],
    out_specs=pl.BlockSpec((_B, D_OUT), lambda i: (i, 0)),
    out_shape=jax.ShapeDtypeStruct((N, D_OUT), jnp.float32),
)


def kernel(x, edge_index, W1, b1, W2, b2):
    row = edge_index[0]
    col = edge_index[1]
    pad_i = jnp.arange(PAD, dtype=jnp.int32)
    rowp = jnp.concatenate([row, pad_i % N])
    colp = jnp.concatenate([col, N + (pad_i % 16)])
    col2 = colp.reshape(ER, 128)
    row2 = rowp.reshape(ER, 128)
    rowpair = jnp.stack([rowp, rowp + N]).reshape(2, ER, 128)
    ones = jnp.ones((NP,), jnp.float32)

    h1 = _tc0(x, W1)
    deg2 = _deg_kernel(ones, col2)
    pdeg = deg2[:, :N].reshape(2, N, 1)
    dis, g1 = _tc1(pdeg, h1)
    agg1 = _agg1(g1.reshape(2 * N, D_HID // 2), rowpair, col2)
    g2 = _tc2(agg1.reshape(2, N, D_HID // 2), dis, b1.reshape(1, D_HID), W2)
    agg2 = _agg2(g2, row2, col2)
    out = _tc3(agg2, g2, dis, b2.reshape(1, D_OUT))
    return out
